# baseline (speedup 1.0000x reference)
import functools

import jax
import jax.numpy as jnp
from jax import lax
from jax.experimental import pallas as pl
from jax.experimental.pallas import tpu as pltpu
from jax.experimental.pallas import tpu_sc as plsc


def _pick_tile(n, target):
    t = min(target, n)
    t = max(8, (t + 7) // 8 * 8)
    return t


_NC, _NS = 2, 16
_NW = _NC * _NS


@functools.lru_cache(maxsize=None)
def _sc_gather_fn(num_idx):
    b = num_idx // _NW
    ch = 1000
    nch = b // ch
    mesh = plsc.VectorSubcoreMesh(core_axis_name="c", subcore_axis_name="s")

    @functools.partial(
        pl.kernel, mesh=mesh,
        out_type=jax.ShapeDtypeStruct((num_idx, 128), jnp.float32),
        scratch_types=[
            pltpu.VMEM((ch,), jnp.int32),
            pltpu.VMEM((ch, 128), jnp.float32),
            pltpu.SemaphoreType.DMA,
        ],
    )
    def k(table_hbm, idx_hbm, out_hbm, idx_v, rows_v, sem):
        wid = lax.axis_index("s") * _NC + lax.axis_index("c")
        base = wid * b
        for c in range(nch):
            pltpu.sync_copy(idx_hbm.at[pl.ds(base + c * ch, ch)], idx_v)
            pltpu.async_copy(table_hbm.at[idx_v], rows_v, sem).wait()
            pltpu.sync_copy(rows_v, out_hbm.at[pl.ds(base + c * ch, ch)])

    return k


def _sc_gather(table, idx):
    n_rows, w = table.shape
    if w < 128:
        table = jnp.pad(table, ((0, 0), (0, 128 - w)))
    return _sc_gather_fn(idx.shape[0])(table, idx)[:, :w]


def _mlp_body(nw, *refs):
    x_ref = refs[0]
    w_refs = refs[1:1 + nw]
    b_refs = refs[1 + nw:1 + 2 * nw]
    o_ref = refs[-1]
    h = x_ref[...]
    for k in range(nw):
        if h.shape[1] == 1:
            h = h * w_refs[k][...] + b_refs[k][...]
        else:
            h = jnp.dot(h, w_refs[k][...],
                        preferred_element_type=jnp.float32) + b_refs[k][...]
        if k < nw - 1:
            h = jnp.maximum(h, 0.0)
    o_ref[...] = h


def _run_mlp(x, ps, tile_target=640):
    n, din = x.shape
    nw = len(ps)
    ws = [w for w, _ in ps]
    bs = [b.reshape(1, -1) for _, b in ps]
    dout = ws[-1].shape[1]
    tile = _pick_tile(n, tile_target)
    npad = -(-n // tile) * tile
    if npad != n:
        x = jnp.pad(x, ((0, npad - n), (0, 0)))
    grid = (npad // tile,)
    in_specs = [pl.BlockSpec((tile, din), lambda i: (i, 0))]
    in_specs += [pl.BlockSpec(w.shape, lambda i: (0, 0)) for w in ws]
    in_specs += [pl.BlockSpec(b.shape, lambda i: (0, 0)) for b in bs]
    out = pl.pallas_call(
        functools.partial(_mlp_body, nw),
        grid=grid,
        in_specs=in_specs,
        out_specs=pl.BlockSpec((tile, dout), lambda i: (i, 0)),
        out_shape=jax.ShapeDtypeStruct((npad, dout), jnp.float32),
    )(x, *ws, *bs)
    return out[:n] if npad != n else out


def _segmax_body(tile, x_ref, seg_ref, mask_ref, o_ref, cval_ref, cseg_ref):
    i = pl.program_id(0)
    neg = jnp.float32(-jnp.inf)
    x = x_ref[...]
    segf = seg_ref[...].astype(jnp.float32)
    m = mask_ref[...]
    val = jnp.where(m > 0, x, neg)

    @pl.when(i == 0)
    def _():
        cval_ref[...] = jnp.full_like(cval_ref, neg)
        cseg_ref[...] = jnp.full_like(cseg_ref, -1.0)

    sh = 1
    while sh < tile:
        xs = jnp.concatenate(
            [jnp.full((sh, x.shape[1]), neg, jnp.float32), val[:-sh]], axis=0)
        ss = jnp.concatenate(
            [jnp.full((sh, 1), -1.0, jnp.float32), segf[:-sh]], axis=0)
        val = jnp.where(ss == segf, jnp.maximum(val, xs), val)
        sh *= 2
    cv = cval_ref[0:1, :]
    cs = cseg_ref[0:1, 0:1]
    val = jnp.where(segf == cs, jnp.maximum(val, cv), val)
    o_ref[...] = val
    cval_ref[0:1, :] = val[tile - 1:tile, :]
    cseg_ref[0:1, 0:1] = segf[tile - 1:tile, 0:1]


def _seg_cummax(x128, seg, mask, tile_target=640):
    n = x128.shape[0]
    tile = _pick_tile(n, tile_target)
    npad = -(-n // tile) * tile
    if npad != n:
        x128 = jnp.pad(x128, ((0, npad - n), (0, 0)))
        seg = jnp.pad(seg, (0, npad - n), constant_values=-2)
        mask = jnp.pad(mask, (0, npad - n))
    out = _seg_cummax_call(x128, seg, mask, tile)
    return out[:n] if npad != n else out


def _seg_cummax_call(x128, seg, mask, tile):
    n = x128.shape[0]
    grid = (n // tile,)
    return pl.pallas_call(
        functools.partial(_segmax_body, tile),
        grid=grid,
        in_specs=[pl.BlockSpec((tile, 128), lambda i: (i, 0)),
                  pl.BlockSpec((tile, 1), lambda i: (i, 0)),
                  pl.BlockSpec((tile, 1), lambda i: (i, 0))],
        out_specs=pl.BlockSpec((tile, 128), lambda i: (i, 0)),
        out_shape=jax.ShapeDtypeStruct((n, 128), jnp.float32),
        scratch_shapes=[pltpu.VMEM((8, 128), jnp.float32),
                        pltpu.VMEM((8, 128), jnp.float32)],
    )(x128, seg.reshape(-1, 1), mask.reshape(-1, 1))


def _stagea_body(x_ref, *refs):
    wm = refs[0:3]
    bm = refs[3:6]
    w = refs[6:9]
    b = refs[9:12]
    wd = refs[12:15]
    bd = refs[15:18]
    msg_ref, mdr_ref = refs[18], refs[19]
    x = x_ref[...]
    xi = x[:, 0:4]
    ev = x[:, 4:8]
    mass = x[:, 8:9]
    dr = x[:, 9:10]
    h = jnp.maximum(mass * wm[0][...] + bm[0][...], 0.0)
    h = jnp.maximum(jnp.dot(h, wm[1][...],
                            preferred_element_type=jnp.float32) + bm[1][...], 0.0)
    score = jnp.dot(h, wm[2][...],
                    preferred_element_type=jnp.float32) + bm[2][...]
    feat = jnp.concatenate([xi, ev, mass, score], axis=1)
    h = jnp.maximum(jnp.dot(feat, w[0][...],
                            preferred_element_type=jnp.float32) + b[0][...], 0.0)
    h = jnp.maximum(jnp.dot(h, w[1][...],
                            preferred_element_type=jnp.float32) + b[1][...], 0.0)
    msgv = jnp.dot(h, w[2][...],
                   preferred_element_type=jnp.float32) + b[2][...]
    msg_ref[...] = jnp.concatenate(
        [msgv, jnp.zeros((msgv.shape[0], 116), jnp.float32)], axis=1)
    h = jnp.maximum(dr * wd[0][...] + bd[0][...], 0.0)
    h = jnp.maximum(jnp.dot(h, wd[1][...],
                            preferred_element_type=jnp.float32) + bd[1][...], 0.0)
    dr12 = jnp.dot(h, wd[2][...],
                   preferred_element_type=jnp.float32) + bd[2][...]
    mdr_ref[...] = jnp.concatenate([score, dr12], axis=1)


def _stage_a(edge10, pm, p, pdr, tile_target=640):
    n = edge10.shape[0]
    ws = ([w for w, _ in pm] + [b.reshape(1, -1) for _, b in pm]
          + [w for w, _ in p] + [b.reshape(1, -1) for _, b in p]
          + [w for w, _ in pdr] + [b.reshape(1, -1) for _, b in pdr])
    tile = _pick_tile(n, tile_target)
    npad = -(-n // tile) * tile
    if npad != n:
        edge10 = jnp.pad(edge10, ((0, npad - n), (0, 0)))
    grid = (npad // tile,)
    in_specs = [pl.BlockSpec((tile, 10), lambda i: (i, 0))]
    in_specs += [pl.BlockSpec(a.shape, lambda i: (0, 0)) for a in ws]
    msg, mdr = pl.pallas_call(
        _stagea_body,
        grid=grid,
        in_specs=in_specs,
        out_specs=(pl.BlockSpec((tile, 128), lambda i: (i, 0)),
                   pl.BlockSpec((tile, 14), lambda i: (i, 0))),
        out_shape=(jax.ShapeDtypeStruct((npad, 128), jnp.float32),
                   jax.ShapeDtypeStruct((npad, 14), jnp.float32)),
    )(edge10, *ws)
    return (msg[:n], mdr[:n]) if npad != n else (msg, mdr)


def _to_pxpypze(pt, eta, phi, e):
    return jnp.concatenate(
        [pt * jnp.cos(phi), pt * jnp.sin(phi), pt * jnp.sinh(eta), e], axis=1)


def _mass(p):
    m2 = p[:, 3:4] ** 2 - p[:, 0:1] ** 2 - p[:, 1:2] ** 2 - p[:, 2:3] ** 2
    return jnp.sqrt(jnp.abs(m2) + 1e-12)


def kernel(edge_index, i, N_pT, N_eta, N_phi, N_energy, G_mu, G_met,
           G_met_phi, G_pileup, G_nTruthJets, params):
    f32 = jnp.float32
    batch_len = i.shape[0]
    n = N_pT.shape[0]
    e0, e1 = edge_index[0], edge_index[1]
    num_e = e0.shape[0]
    use_sc = (num_e % (8 * _NW) == 0) and (num_e // _NW) % 1000 == 0

    def gather(table, idx):
        if use_sc:
            return _sc_gather(table, idx)
        return table[idx]

    P_mu = _to_pxpypze(N_pT, N_eta, N_phi, N_energy)

    node_tab = jnp.concatenate([P_mu, N_eta, N_phi], axis=1)
    g0 = gather(node_tab, e0)
    g1 = gather(node_tab, e1)
    x_j, x_i = g0[:, 0:4], g1[:, 0:4]
    ev = x_i + x_j
    edge_mass = _mass(ev)
    dr = jnp.sqrt((g0[:, 4:5] - g1[:, 4:5]) ** 2
                  + (g0[:, 5:6] - g1[:, 5:6]) ** 2 + 1e-12)
    edge10 = jnp.concatenate([x_i, ev, edge_mass, dr], axis=1)
    msg, mdr = _stage_a(edge10, params['mlp_m'], params['mlp'],
                        params['mlp_dr'])

    iota_e = jnp.arange(num_e, dtype=jnp.int32)
    e1_s, perm1, e0_s1 = lax.sort((e1, iota_e, e0), num_keys=1)
    ar_n = jnp.arange(n, dtype=e1_s.dtype)
    right1 = jnp.searchsorted(e1_s, ar_n, side='right')
    has1 = right1 > jnp.searchsorted(e1_s, ar_n, side='left')
    end1 = jnp.where(has1, right1 - 1, 0).astype(jnp.int32)
    idx_pe = end1[e0]
    valid_pe = has1[e0]
    msg_s = gather(msg, perm1)

    mask = jnp.ones((num_e,), dtype=f32)
    mask_s1 = jnp.ones((num_e,), dtype=f32)
    o_topo = None
    g_acc = None
    g_acc_s1 = None
    for it in range(3):
        cmx = _seg_cummax(msg_s, e1_s, mask_s1)
        pe = gather(cmx, idx_pe)[:, :12]
        prop_e = jnp.where(valid_pe[:, None] & jnp.isfinite(pe), pe, 0.0)
        upd = _run_mlp(jnp.concatenate([prop_e, mdr], axis=1),
                       params['mlp_edge'], tile_target=1600)
        if it == 0:
            o_topo = upd
            head = upd[:n]
            g_acc = gather(head, e0)
            g_acc_s1 = gather(head, e0_s1)
        else:
            delta = jax.ops.segment_sum(mask[:, None] * upd, e0,
                                        num_segments=n)
            o_topo = o_topo + jnp.pad(delta, ((0, num_e - n), (0, 0)))
            g_acc = g_acc + gather(delta, e0)
            g_acc_s1 = g_acc_s1 + gather(delta, e0_s1)
        mask = mask * (g_acc[:, 1] > g_acc[:, 0]).astype(f32)
        mask_s1 = mask_s1 * (g_acc_s1[:, 1] > g_acc_s1[:, 0]).astype(f32)

    e_i_active = jnp.argmax(o_topo, axis=1)
    sel = (e_i_active == 1).astype(f32)
    P_mu_p = jax.ops.segment_sum(sel[:, None] * x_i, e0, num_segments=n)
    t_m = _mass(P_mu_p)
    t_ = t_m.reshape(batch_len, -1)
    ts = jnp.sort(t_, axis=1)
    pos = ts > 0
    newv = jnp.concatenate(
        [pos[:, :1], (ts[:, 1:] != ts[:, :-1]) & pos[:, 1:]], axis=1)
    nTops = newv.sum(axis=1).astype(f32).reshape(-1, 1)

    P_mu_b = P_mu.reshape(batch_len, -1, 4).sum(axis=1)
    px, py = P_mu_b[:, 0:1], P_mu_b[:, 1:2]
    pt_b = jnp.sqrt(px ** 2 + py ** 2 + 1e-12)
    MET_Meas = -pt_b
    MET_Phi = -jnp.arctan2(py, px)

    mu_in = jnp.concatenate(
        [nTops, G_mu, G_pileup, MET_Meas - G_met, MET_Phi - G_met_phi,
         G_nTruthJets], axis=1)
    O_mu_actual = _run_mlp(mu_in, params['mlp_mu'], tile_target=128)
    nt_in = jnp.concatenate([nTops, MET_Meas, MET_Phi, G_nTruthJets], axis=1)
    O_nTops = _run_mlp(nt_in, params['mlp_ntops'], tile_target=128)

    topo_in = jnp.concatenate(
        [o_topo, e_i_active.reshape(-1, 1).astype(f32), gather(P_mu_p, e0),
         gather(P_mu_p, e1)], axis=1)
    O_Topo = _run_mlp(topo_in, params['mlp_topo'])
    idx_in = jnp.concatenate([t_m, P_mu_p - P_mu], axis=1)
    O_Index = _run_mlp(idx_in, params['mlp_mnodetops'], tile_target=1000)
    return (O_Topo, O_mu_actual, O_nTops, O_Index)


# speedup vs baseline: 1.4308x; 1.4308x over previous
import functools

import jax
import jax.numpy as jnp
from jax import lax
from jax.experimental import pallas as pl
from jax.experimental.pallas import tpu as pltpu
from jax.experimental.pallas import tpu_sc as plsc


def _pick_tile(n, target):
    t = min(target, n)
    t = max(8, (t + 7) // 8 * 8)
    return t


_NC, _NS = 2, 16
_NW = _NC * _NS


@functools.lru_cache(maxsize=None)
def _sc_gather_fn(num_idx):
    b = num_idx // _NW
    ch = 1000
    nch = b // ch
    mesh = plsc.VectorSubcoreMesh(core_axis_name="c", subcore_axis_name="s")

    @functools.partial(
        pl.kernel, mesh=mesh,
        out_type=jax.ShapeDtypeStruct((num_idx, 128), jnp.float32),
        scratch_types=[
            pltpu.VMEM((ch,), jnp.int32),
            pltpu.VMEM((ch, 128), jnp.float32),
            pltpu.SemaphoreType.DMA,
        ],
    )
    def k(table_hbm, idx_hbm, out_hbm, idx_v, rows_v, sem):
        wid = lax.axis_index("s") * _NC + lax.axis_index("c")
        base = wid * b
        for c in range(nch):
            pltpu.sync_copy(idx_hbm.at[pl.ds(base + c * ch, ch)], idx_v)
            pltpu.async_copy(table_hbm.at[idx_v], rows_v, sem).wait()
            pltpu.sync_copy(rows_v, out_hbm.at[pl.ds(base + c * ch, ch)])

    return k


def _sc_gather(table, idx):
    n_rows, w = table.shape
    if w < 128:
        table = jnp.pad(table, ((0, 0), (0, 128 - w)))
    return _sc_gather_fn(idx.shape[0])(table, idx)[:, :w]


def _mlp_body(nw, *refs):
    x_ref = refs[0]
    w_refs = refs[1:1 + nw]
    b_refs = refs[1 + nw:1 + 2 * nw]
    o_ref = refs[-1]
    h = x_ref[...]
    for k in range(nw):
        if h.shape[1] == 1:
            h = h * w_refs[k][...] + b_refs[k][...]
        else:
            h = jnp.dot(h, w_refs[k][...],
                        preferred_element_type=jnp.float32) + b_refs[k][...]
        if k < nw - 1:
            h = jnp.maximum(h, 0.0)
    o_ref[...] = h


def _run_mlp(x, ps, tile_target=640):
    n, din = x.shape
    nw = len(ps)
    ws = [w for w, _ in ps]
    bs = [b.reshape(1, -1) for _, b in ps]
    dout = ws[-1].shape[1]
    tile = _pick_tile(n, tile_target)
    npad = -(-n // tile) * tile
    if npad != n:
        x = jnp.pad(x, ((0, npad - n), (0, 0)))
    grid = (npad // tile,)
    in_specs = [pl.BlockSpec((tile, din), lambda i: (i, 0))]
    in_specs += [pl.BlockSpec(w.shape, lambda i: (0, 0)) for w in ws]
    in_specs += [pl.BlockSpec(b.shape, lambda i: (0, 0)) for b in bs]
    out = pl.pallas_call(
        functools.partial(_mlp_body, nw),
        grid=grid,
        in_specs=in_specs,
        out_specs=pl.BlockSpec((tile, dout), lambda i: (i, 0)),
        out_shape=jax.ShapeDtypeStruct((npad, dout), jnp.float32),
    )(x, *ws, *bs)
    return out[:n] if npad != n else out


def _stagea_body(x_ref, *refs):
    wm = refs[0:3]
    bm = refs[3:6]
    w = refs[6:9]
    b = refs[9:12]
    wd = refs[12:15]
    bd = refs[15:18]
    msg_ref, mdr_ref = refs[18], refs[19]
    x = x_ref[...]
    xi = x[:, 0:4]
    ev = x[:, 4:8]
    mass = x[:, 8:9]
    dr = x[:, 9:10]
    h = jnp.maximum(mass * wm[0][...] + bm[0][...], 0.0)
    h = jnp.maximum(jnp.dot(h, wm[1][...],
                            preferred_element_type=jnp.float32) + bm[1][...], 0.0)
    score = jnp.dot(h, wm[2][...],
                    preferred_element_type=jnp.float32) + bm[2][...]
    feat = jnp.concatenate([xi, ev, mass, score], axis=1)
    h = jnp.maximum(jnp.dot(feat, w[0][...],
                            preferred_element_type=jnp.float32) + b[0][...], 0.0)
    h = jnp.maximum(jnp.dot(h, w[1][...],
                            preferred_element_type=jnp.float32) + b[1][...], 0.0)
    msg_ref[...] = jnp.dot(h, w[2][...],
                           preferred_element_type=jnp.float32) + b[2][...]
    h = jnp.maximum(dr * wd[0][...] + bd[0][...], 0.0)
    h = jnp.maximum(jnp.dot(h, wd[1][...],
                            preferred_element_type=jnp.float32) + bd[1][...], 0.0)
    dr12 = jnp.dot(h, wd[2][...],
                   preferred_element_type=jnp.float32) + bd[2][...]
    mdr_ref[...] = jnp.concatenate([score, dr12], axis=1)


def _stage_a(edge10, pm, p, pdr, tile_target=640):
    n = edge10.shape[0]
    ws = ([w for w, _ in pm] + [b.reshape(1, -1) for _, b in pm]
          + [w for w, _ in p] + [b.reshape(1, -1) for _, b in p]
          + [w for w, _ in pdr] + [b.reshape(1, -1) for _, b in pdr])
    tile = _pick_tile(n, tile_target)
    npad = -(-n // tile) * tile
    if npad != n:
        edge10 = jnp.pad(edge10, ((0, npad - n), (0, 0)))
    grid = (npad // tile,)
    in_specs = [pl.BlockSpec((tile, 10), lambda i: (i, 0))]
    in_specs += [pl.BlockSpec(a.shape, lambda i: (0, 0)) for a in ws]
    msg, mdr = pl.pallas_call(
        _stagea_body,
        grid=grid,
        in_specs=in_specs,
        out_specs=(pl.BlockSpec((tile, 12), lambda i: (i, 0)),
                   pl.BlockSpec((tile, 14), lambda i: (i, 0))),
        out_shape=(jax.ShapeDtypeStruct((npad, 12), jnp.float32),
                   jax.ShapeDtypeStruct((npad, 14), jnp.float32)),
    )(edge10, *ws)
    return (msg[:n], mdr[:n]) if npad != n else (msg, mdr)


def _to_pxpypze(pt, eta, phi, e):
    return jnp.concatenate(
        [pt * jnp.cos(phi), pt * jnp.sin(phi), pt * jnp.sinh(eta), e], axis=1)


def _mass(p):
    m2 = p[:, 3:4] ** 2 - p[:, 0:1] ** 2 - p[:, 1:2] ** 2 - p[:, 2:3] ** 2
    return jnp.sqrt(jnp.abs(m2) + 1e-12)


def kernel(edge_index, i, N_pT, N_eta, N_phi, N_energy, G_mu, G_met,
           G_met_phi, G_pileup, G_nTruthJets, params):
    f32 = jnp.float32
    batch_len = i.shape[0]
    n = N_pT.shape[0]
    e0, e1 = edge_index[0], edge_index[1]
    num_e = e0.shape[0]
    use_sc = (num_e % (8 * _NW) == 0) and (num_e // _NW) % 1000 == 0

    def gather(table, idx):
        if use_sc:
            return _sc_gather(table, idx)
        return table[idx]

    def gather2(table, idx_a, idx_b):
        if not use_sc:
            return table[idx_a], table[idx_b]
        ne_a = idx_a.shape[0]
        out = _sc_gather(table, jnp.concatenate([idx_a, idx_b]))
        return out[:ne_a], out[ne_a:]

    P_mu = _to_pxpypze(N_pT, N_eta, N_phi, N_energy)

    node_tab = jnp.concatenate([P_mu, N_eta, N_phi], axis=1)
    g0, g1 = gather2(node_tab, e0, e1)
    x_j, x_i = g0[:, 0:4], g1[:, 0:4]
    ev = x_i + x_j
    edge_mass = _mass(ev)
    dr = jnp.sqrt((g0[:, 4:5] - g1[:, 4:5]) ** 2
                  + (g0[:, 5:6] - g1[:, 5:6]) ** 2 + 1e-12)
    edge10 = jnp.concatenate([x_i, ev, edge_mass, dr], axis=1)
    msg, mdr = _stage_a(edge10, params['mlp_m'], params['mlp'],
                        params['mlp_dr'])

    mask = jnp.ones((num_e,), dtype=f32)
    o_topo = None
    g_acc = None
    for it in range(3):
        mm = jnp.where(mask[:, None] > 0, msg, -jnp.inf)
        prop = jax.ops.segment_max(mm, e1, num_segments=n)
        prop = jnp.where(jnp.isfinite(prop), prop, 0.0)
        upd = _run_mlp(jnp.concatenate([gather(prop, e0), mdr], axis=1),
                       params['mlp_edge'], tile_target=1600)
        if it == 0:
            o_topo = upd
            g_acc = gather(upd[:n], e0)
        else:
            delta = jax.ops.segment_sum(mask[:, None] * upd, e0,
                                        num_segments=n)
            o_topo = o_topo + jnp.pad(delta, ((0, num_e - n), (0, 0)))
            g_acc = g_acc + gather(delta, e0)
        mask = mask * (g_acc[:, 1] > g_acc[:, 0]).astype(f32)

    e_i_active = jnp.argmax(o_topo, axis=1)
    sel = (e_i_active == 1).astype(f32)
    P_mu_p = jax.ops.segment_sum(sel[:, None] * x_i, e0, num_segments=n)
    t_m = _mass(P_mu_p)
    t_ = t_m.reshape(batch_len, -1)
    ts = jnp.sort(t_, axis=1)
    pos = ts > 0
    newv = jnp.concatenate(
        [pos[:, :1], (ts[:, 1:] != ts[:, :-1]) & pos[:, 1:]], axis=1)
    nTops = newv.sum(axis=1).astype(f32).reshape(-1, 1)

    P_mu_b = P_mu.reshape(batch_len, -1, 4).sum(axis=1)
    px, py = P_mu_b[:, 0:1], P_mu_b[:, 1:2]
    pt_b = jnp.sqrt(px ** 2 + py ** 2 + 1e-12)
    MET_Meas = -pt_b
    MET_Phi = -jnp.arctan2(py, px)

    mu_in = jnp.concatenate(
        [nTops, G_mu, G_pileup, MET_Meas - G_met, MET_Phi - G_met_phi,
         G_nTruthJets], axis=1)
    O_mu_actual = _run_mlp(mu_in, params['mlp_mu'], tile_target=128)
    nt_in = jnp.concatenate([nTops, MET_Meas, MET_Phi, G_nTruthJets], axis=1)
    O_nTops = _run_mlp(nt_in, params['mlp_ntops'], tile_target=128)

    pp0, pp1 = gather2(P_mu_p, e0, e1)
    topo_in = jnp.concatenate(
        [o_topo, e_i_active.reshape(-1, 1).astype(f32), pp0, pp1], axis=1)
    O_Topo = _run_mlp(topo_in, params['mlp_topo'])
    idx_in = jnp.concatenate([t_m, P_mu_p - P_mu], axis=1)
    O_Index = _run_mlp(idx_in, params['mlp_mnodetops'], tile_target=1000)
    return (O_Topo, O_mu_actual, O_nTops, O_Index)


# speedup vs baseline: 1.4549x; 1.0168x over previous
import functools

import jax
import jax.numpy as jnp
from jax import lax
from jax.experimental import pallas as pl
from jax.experimental.pallas import tpu as pltpu
from jax.experimental.pallas import tpu_sc as plsc


def _pick_tile(n, target):
    t = min(target, n)
    t = max(8, (t + 7) // 8 * 8)
    return t


_NC, _NS = 2, 16
_NW = _NC * _NS


@functools.lru_cache(maxsize=None)
def _sc_gather_fn(num_idx):
    b = num_idx // _NW
    ch = 1000
    nch = b // ch
    mesh = plsc.VectorSubcoreMesh(core_axis_name="c", subcore_axis_name="s")

    @functools.partial(
        pl.kernel, mesh=mesh,
        out_type=jax.ShapeDtypeStruct((num_idx, 128), jnp.float32),
        scratch_types=[
            pltpu.VMEM((ch,), jnp.int32),
            pltpu.VMEM((ch, 128), jnp.float32),
            pltpu.SemaphoreType.DMA,
        ],
    )
    def k(table_hbm, idx_hbm, out_hbm, idx_v, rows_v, sem):
        wid = lax.axis_index("s") * _NC + lax.axis_index("c")
        base = wid * b
        for c in range(nch):
            pltpu.sync_copy(idx_hbm.at[pl.ds(base + c * ch, ch)], idx_v)
            pltpu.async_copy(table_hbm.at[idx_v], rows_v, sem).wait()
            pltpu.sync_copy(rows_v, out_hbm.at[pl.ds(base + c * ch, ch)])

    return k


def _sc_gather(table, idx):
    n_rows, w = table.shape
    if w < 128:
        table = jnp.pad(table, ((0, 0), (0, 128 - w)))
    return _sc_gather_fn(idx.shape[0])(table, idx)[:, :w]


def _mlp_body(nw, *refs):
    x_ref = refs[0]
    w_refs = refs[1:1 + nw]
    b_refs = refs[1 + nw:1 + 2 * nw]
    o_ref = refs[-1]
    h = x_ref[...]
    for k in range(nw):
        if h.shape[1] == 1:
            h = h * w_refs[k][...] + b_refs[k][...]
        else:
            h = jnp.dot(h, w_refs[k][...],
                        preferred_element_type=jnp.float32) + b_refs[k][...]
        if k < nw - 1:
            h = jnp.maximum(h, 0.0)
    o_ref[...] = h


def _run_mlp(x, ps, tile_target=640):
    n, din = x.shape
    nw = len(ps)
    ws = [w for w, _ in ps]
    bs = [b.reshape(1, -1) for _, b in ps]
    dout = ws[-1].shape[1]
    tile = _pick_tile(n, tile_target)
    npad = -(-n // tile) * tile
    if npad != n:
        x = jnp.pad(x, ((0, npad - n), (0, 0)))
    grid = (npad // tile,)
    in_specs = [pl.BlockSpec((tile, din), lambda i: (i, 0))]
    in_specs += [pl.BlockSpec(w.shape, lambda i: (0, 0)) for w in ws]
    in_specs += [pl.BlockSpec(b.shape, lambda i: (0, 0)) for b in bs]
    out = pl.pallas_call(
        functools.partial(_mlp_body, nw),
        grid=grid,
        in_specs=in_specs,
        out_specs=pl.BlockSpec((tile, dout), lambda i: (i, 0)),
        out_shape=jax.ShapeDtypeStruct((npad, dout), jnp.float32),
    )(x, *ws, *bs)
    return out[:n] if npad != n else out


def _stagea_body(x_ref, *refs):
    wm = refs[0:3]
    bm = refs[3:6]
    w = refs[6:9]
    b = refs[9:12]
    wd = refs[12:15]
    bd = refs[15:18]
    msg_ref, mdr_ref = refs[18], refs[19]
    x = x_ref[...]
    xi = x[:, 0:4]
    ev = x[:, 4:8]
    mass = x[:, 8:9]
    dr = x[:, 9:10]
    h = jnp.maximum(mass * wm[0][...] + bm[0][...], 0.0)
    h = jnp.maximum(jnp.dot(h, wm[1][...],
                            preferred_element_type=jnp.float32) + bm[1][...], 0.0)
    score = jnp.dot(h, wm[2][...],
                    preferred_element_type=jnp.float32) + bm[2][...]
    feat = jnp.concatenate([xi, ev, mass, score], axis=1)
    h = jnp.maximum(jnp.dot(feat, w[0][...],
                            preferred_element_type=jnp.float32) + b[0][...], 0.0)
    h = jnp.maximum(jnp.dot(h, w[1][...],
                            preferred_element_type=jnp.float32) + b[1][...], 0.0)
    msg_ref[...] = jnp.dot(h, w[2][...],
                           preferred_element_type=jnp.float32) + b[2][...]
    h = jnp.maximum(dr * wd[0][...] + bd[0][...], 0.0)
    h = jnp.maximum(jnp.dot(h, wd[1][...],
                            preferred_element_type=jnp.float32) + bd[1][...], 0.0)
    dr12 = jnp.dot(h, wd[2][...],
                   preferred_element_type=jnp.float32) + bd[2][...]
    mdr_ref[...] = jnp.concatenate([score, dr12], axis=1)


def _stage_a(edge10, pm, p, pdr, tile_target=640):
    n = edge10.shape[0]
    ws = ([w for w, _ in pm] + [b.reshape(1, -1) for _, b in pm]
          + [w for w, _ in p] + [b.reshape(1, -1) for _, b in p]
          + [w for w, _ in pdr] + [b.reshape(1, -1) for _, b in pdr])
    tile = _pick_tile(n, tile_target)
    npad = -(-n // tile) * tile
    if npad != n:
        edge10 = jnp.pad(edge10, ((0, npad - n), (0, 0)))
    grid = (npad // tile,)
    in_specs = [pl.BlockSpec((tile, 10), lambda i: (i, 0))]
    in_specs += [pl.BlockSpec(a.shape, lambda i: (0, 0)) for a in ws]
    msg, mdr = pl.pallas_call(
        _stagea_body,
        grid=grid,
        in_specs=in_specs,
        out_specs=(pl.BlockSpec((tile, 12), lambda i: (i, 0)),
                   pl.BlockSpec((tile, 14), lambda i: (i, 0))),
        out_shape=(jax.ShapeDtypeStruct((npad, 12), jnp.float32),
                   jax.ShapeDtypeStruct((npad, 14), jnp.float32)),
    )(edge10, *ws)
    return (msg[:n], mdr[:n]) if npad != n else (msg, mdr)


def _to_pxpypze(pt, eta, phi, e):
    return jnp.concatenate(
        [pt * jnp.cos(phi), pt * jnp.sin(phi), pt * jnp.sinh(eta), e], axis=1)


def _mass(p):
    m2 = p[:, 3:4] ** 2 - p[:, 0:1] ** 2 - p[:, 1:2] ** 2 - p[:, 2:3] ** 2
    return jnp.sqrt(jnp.abs(m2) + 1e-12)


def kernel(edge_index, i, N_pT, N_eta, N_phi, N_energy, G_mu, G_met,
           G_met_phi, G_pileup, G_nTruthJets, params):
    f32 = jnp.float32
    batch_len = i.shape[0]
    n = N_pT.shape[0]
    e0, e1 = edge_index[0], edge_index[1]
    num_e = e0.shape[0]
    use_sc = (num_e % (8 * _NW) == 0) and (num_e // _NW) % 1000 == 0

    def gather(table, idx):
        if use_sc:
            return _sc_gather(table, idx)
        return table[idx]

    P_mu = _to_pxpypze(N_pT, N_eta, N_phi, N_energy)

    node_tab = jnp.concatenate([P_mu, N_eta, N_phi], axis=1)
    g0 = gather(node_tab, e0)
    g1 = gather(node_tab, e1)
    x_j, x_i = g0[:, 0:4], g1[:, 0:4]
    ev = x_i + x_j
    edge_mass = _mass(ev)
    dr = jnp.sqrt((g0[:, 4:5] - g1[:, 4:5]) ** 2
                  + (g0[:, 5:6] - g1[:, 5:6]) ** 2 + 1e-12)
    edge10 = jnp.concatenate([x_i, ev, edge_mass, dr], axis=1)
    msg, mdr = _stage_a(edge10, params['mlp_m'], params['mlp'],
                        params['mlp_dr'])

    mask = jnp.ones((num_e,), dtype=f32)
    o_topo = None
    g_acc = None
    for it in range(3):
        mm = jnp.where(mask[:, None] > 0, msg, -jnp.inf)
        prop = jax.ops.segment_max(mm, e1, num_segments=n)
        prop = jnp.where(jnp.isfinite(prop), prop, 0.0)
        upd = _run_mlp(jnp.concatenate([gather(prop, e0), mdr], axis=1),
                       params['mlp_edge'], tile_target=1600)
        if it == 0:
            o_topo = upd
            g_acc = gather(upd[:n], e0)
        else:
            delta = jax.ops.segment_sum(mask[:, None] * upd, e0,
                                        num_segments=n)
            o_topo = o_topo + jnp.pad(delta, ((0, num_e - n), (0, 0)))
            g_acc = g_acc + gather(delta, e0)
        mask = mask * (g_acc[:, 1] > g_acc[:, 0]).astype(f32)

    e_i_active = jnp.argmax(o_topo, axis=1)
    sel = (e_i_active == 1).astype(f32)
    P_mu_p = jax.ops.segment_sum(sel[:, None] * x_i, e0, num_segments=n)
    t_m = _mass(P_mu_p)
    t_ = t_m.reshape(batch_len, -1)
    ts = jnp.sort(t_, axis=1)
    pos = ts > 0
    newv = jnp.concatenate(
        [pos[:, :1], (ts[:, 1:] != ts[:, :-1]) & pos[:, 1:]], axis=1)
    nTops = newv.sum(axis=1).astype(f32).reshape(-1, 1)

    P_mu_b = P_mu.reshape(batch_len, -1, 4).sum(axis=1)
    px, py = P_mu_b[:, 0:1], P_mu_b[:, 1:2]
    pt_b = jnp.sqrt(px ** 2 + py ** 2 + 1e-12)
    MET_Meas = -pt_b
    MET_Phi = -jnp.arctan2(py, px)

    mu_in = jnp.concatenate(
        [nTops, G_mu, G_pileup, MET_Meas - G_met, MET_Phi - G_met_phi,
         G_nTruthJets], axis=1)
    O_mu_actual = _run_mlp(mu_in, params['mlp_mu'], tile_target=128)
    nt_in = jnp.concatenate([nTops, MET_Meas, MET_Phi, G_nTruthJets], axis=1)
    O_nTops = _run_mlp(nt_in, params['mlp_ntops'], tile_target=128)

    topo_in = jnp.concatenate(
        [o_topo, e_i_active.reshape(-1, 1).astype(f32), gather(P_mu_p, e0),
         gather(P_mu_p, e1)], axis=1)
    O_Topo = _run_mlp(topo_in, params['mlp_topo'])
    idx_in = jnp.concatenate([t_m, P_mu_p - P_mu], axis=1)
    O_Index = _run_mlp(idx_in, params['mlp_mnodetops'], tile_target=1000)
    return (O_Topo, O_mu_actual, O_nTops, O_Index)
